# Initial kernel scaffold; baseline (speedup 1.0000x reference)
import jax
import jax.numpy as jnp
from jax.experimental import pallas as pl


def kernel(x, edge_index, W1, b1, W2, b2):
    raise NotImplementedError("write your pallas kernel here")


# speedup vs baseline: 12.0439x; 12.0439x over previous
import functools

import jax
import jax.numpy as jnp
from jax import lax
from jax.experimental import pallas as pl
from jax.experimental.pallas import tpu as pltpu
from jax.experimental.pallas import tpu_sc as plsc

N = 10000
E = 320000
D_IN = 128
D_H = 256
D_OUT = 128

NPAD = 10240
NSUB = 16
NCORE = 2
NW = NSUB * NCORE
RPT = NPAD // NSUB
CK = 128
NCH2 = 79
NCH1 = 2 * NCH2
TOTCH = NW * NCH2
EPAD = TOTCH * CK
BLK = 1024

_mesh = plsc.VectorSubcoreMesh(
    core_axis_name="c", subcore_axis_name="s",
    num_cores=NCORE, num_subcores=NSUB)


def _zero_rows(buf, nrow, ncol):
    def row(i, _):
        def col(k, _):
            buf[i, pl.ds(k * 16, 16)] = jnp.zeros((16,), jnp.float32)
            return 0
        lax.fori_loop(0, ncol // 16, col, 0)
        return 0
    lax.fori_loop(0, nrow, row, 0)


@functools.partial(
    pl.kernel,
    out_type=jax.ShapeDtypeStruct((NPAD,), jnp.float32),
    mesh=_mesh,
    scratch_types=[
        pltpu.VMEM_SHARED((NPAD,), jnp.float32),
        pltpu.VMEM((NCH1, CK), jnp.int32),
        pltpu.VMEM((CK,), jnp.float32),
        pltpu.VMEM((RPT,), jnp.float32),
    ],
)
def _deg_kernel(dst_hbm, deg_hbm, deg_acc, didx, ones, wbuf):
    c = lax.axis_index("c")
    s = lax.axis_index("s")
    r0 = s * RPT

    def fill(i, _):
        wbuf[pl.ds(i * 16, 16)] = jnp.zeros((16,), jnp.float32)
        return 0
    lax.fori_loop(0, RPT // 16, fill, 0)

    def fill1(i, _):
        ones[pl.ds(i * 16, 16)] = jnp.ones((16,), jnp.float32)
        return 0
    lax.fori_loop(0, CK // 16, fill1, 0)

    pltpu.sync_copy(wbuf, deg_acc.at[pl.ds(r0, RPT)])
    pltpu.sync_copy(dst_hbm.at[s], didx)
    plsc.subcore_barrier()

    def scat(j, _):
        pltpu.sync_copy(ones, deg_acc.at[didx.at[j]], add=True)
        return 0
    lax.fori_loop(0, NCH1, scat, 0)
    plsc.subcore_barrier()

    @pl.when(c == 0)
    def _():
        pltpu.sync_copy(deg_acc.at[pl.ds(r0, RPT)], deg_hbm.at[pl.ds(r0, RPT)])


def _make_agg_kernel(col_split):
    nch = NCH1 if col_split else NCH2
    rbufs = 2
    ring = 4

    @functools.partial(
        pl.kernel,
        out_type=jax.ShapeDtypeStruct((2 * NPAD, 128), jnp.float32),
        mesh=_mesh,
        scratch_types=[
            pltpu.VMEM_SHARED((NPAD, 128), jnp.float32),
            pltpu.VMEM((ring, 2, CK), jnp.int32),
            pltpu.VMEM((CK, 128), jnp.float32),
            pltpu.VMEM((CK, 128), jnp.float32),
            pltpu.SemaphoreType.DMA,
            pltpu.SemaphoreType.DMA,
            pltpu.SemaphoreType.DMA,
            pltpu.SemaphoreType.DMA,
            pltpu.SemaphoreType.DMA,
            pltpu.SemaphoreType.DMA,
        ],
    )
    def _agg(g_hbm, esd_hbm, agg_hbm, acc, idxr, rb0, rb1,
             si0, si1, si2, si3, sem0, sem1):
        semis = (si0, si1, si2, si3)
        c = lax.axis_index("c")
        s = lax.axis_index("s")
        r0 = s * RPT
        if col_split:
            off = c * NPAD
            cb = s * nch
        else:
            off = 0
            cb = (s * NCORE + c) * nch
        rbs = (rb0, rb1)
        sems = (sem0, sem1)

        _zero_rows(rb0, CK, 128)

        def zacc(k, _):
            pltpu.sync_copy(rb0, acc.at[pl.ds(r0 + k * CK, CK)])
            return 0
        lax.fori_loop(0, RPT // CK, zacc, 0)

        def offset_src(u):
            if col_split:
                for k in range(CK // 16):
                    idxr[u, 0, pl.ds(k * 16, 16)] = (
                        idxr[u, 0, pl.ds(k * 16, 16)] + off)

        plsc.subcore_barrier()

        for u in range(ring):
            pltpu.async_copy(esd_hbm.at[cb + u], idxr.at[u], semis[u])
        for u in range(rbufs):
            pltpu.make_async_copy(
                esd_hbm.at[cb + u], idxr.at[u], semis[u]).wait()
            offset_src(u)
            pltpu.async_copy(g_hbm.at[idxr.at[u, 0]], rbs[u], sems[u])

        def substep(j, u):
            b = u % rbufs
            pltpu.make_async_copy(
                g_hbm.at[idxr.at[u, 0]], rbs[b], sems[b]).wait()
            pltpu.sync_copy(rbs[b], acc.at[idxr.at[u, 1]], add=True)

            @pl.when(j + ring < nch)
            def _():
                pltpu.async_copy(
                    esd_hbm.at[cb + j + ring], idxr.at[u], semis[u])

            u2 = (u + rbufs) % ring

            @pl.when(j + rbufs < nch)
            def _():
                pltpu.make_async_copy(
                    esd_hbm.at[cb + j + rbufs], idxr.at[u2], semis[u2]).wait()
                offset_src(u2)
                pltpu.async_copy(g_hbm.at[idxr.at[u2, 0]], rbs[b], sems[b])

        def step(k, _):
            for u in range(ring):
                substep(ring * k + u, u)
            return 0
        lax.fori_loop(0, nch // ring, step, 0)
        for u in range(nch % ring):
            substep(ring * (nch // ring) + u, u)

        plsc.subcore_barrier()

        def wb(k, _):
            pltpu.sync_copy(acc.at[pl.ds(r0 + k * CK, CK)], rb0)
            pltpu.sync_copy(
                rb0, agg_hbm.at[pl.ds(c * NPAD + r0 + k * CK, CK)])
            return 0
        lax.fori_loop(0, RPT // CK, wb, 0)

    return _agg


_agg_l1 = _make_agg_kernel(col_split=True)
_agg_l2 = _make_agg_kernel(col_split=False)


def _elu(v):
    return jnp.where(v > 0.0, v, jnp.exp(v) - 1.0)


def _mm1_body(x_ref, w_ref, deg_ref, o_ref):
    di = lax.rsqrt(deg_ref[...] + 1.0)
    o_ref[...] = jnp.dot(x_ref[...], w_ref[...],
                         preferred_element_type=jnp.float32) * di


def _l2_body(a0, a1, g0, g1, deg_ref, b1_ref, w2_ref, o_ref):
    di = lax.rsqrt(deg_ref[...] + 1.0)
    h0 = _elu(di * (a0[...] + g0[...]) + b1_ref[0:1, 0:128])
    h1 = _elu(di * (a1[...] + g1[...]) + b1_ref[0:1, 128:256])
    h = jnp.concatenate([h0, h1], axis=1)
    o_ref[...] = jnp.dot(h, w2_ref[...],
                         preferred_element_type=jnp.float32) * di


def _fin_body(p0, p1, g2, deg_ref, b2_ref, o_ref):
    di = lax.rsqrt(deg_ref[...] + 1.0)
    o_ref[...] = _elu(di * (p0[...] + p1[...] + g2[...]) + b2_ref[...])


def _mm1(x_pad, w1, deg2d):
    nb = NPAD // BLK
    return pl.pallas_call(
        _mm1_body,
        grid=(nb, 2),
        in_specs=[
            pl.BlockSpec((BLK, D_IN), lambda i, c: (i, 0)),
            pl.BlockSpec((D_IN, 128), lambda i, c: (0, c)),
            pl.BlockSpec((BLK, 1), lambda i, c: (i, 0)),
        ],
        out_specs=pl.BlockSpec((BLK, 128), lambda i, c: (c * nb + i, 0)),
        out_shape=jax.ShapeDtypeStruct((2 * NPAD, 128), jnp.float32),
    )(x_pad, w1, deg2d)


def _layer2(agg1, g1, deg2d, b1row, w2):
    nb = NPAD // BLK
    return pl.pallas_call(
        _l2_body,
        grid=(nb,),
        in_specs=[
            pl.BlockSpec((BLK, 128), lambda i: (i, 0)),
            pl.BlockSpec((BLK, 128), lambda i: (nb + i, 0)),
            pl.BlockSpec((BLK, 128), lambda i: (i, 0)),
            pl.BlockSpec((BLK, 128), lambda i: (nb + i, 0)),
            pl.BlockSpec((BLK, 1), lambda i: (i, 0)),
            pl.BlockSpec((1, D_H), lambda i: (0, 0)),
            pl.BlockSpec((D_H, D_OUT), lambda i: (0, 0)),
        ],
        out_specs=pl.BlockSpec((BLK, D_OUT), lambda i: (i, 0)),
        out_shape=jax.ShapeDtypeStruct((NPAD, D_OUT), jnp.float32),
    )(agg1, agg1, g1, g1, deg2d, b1row, w2)


def _final(agg2, g2, deg2d, b2row):
    nb = NPAD // BLK
    return pl.pallas_call(
        _fin_body,
        grid=(nb,),
        in_specs=[
            pl.BlockSpec((BLK, D_OUT), lambda i: (i, 0)),
            pl.BlockSpec((BLK, D_OUT), lambda i: (nb + i, 0)),
            pl.BlockSpec((BLK, D_OUT), lambda i: (i, 0)),
            pl.BlockSpec((BLK, 1), lambda i: (i, 0)),
            pl.BlockSpec((1, D_OUT), lambda i: (0, 0)),
        ],
        out_specs=pl.BlockSpec((BLK, D_OUT), lambda i: (i, 0)),
        out_shape=jax.ShapeDtypeStruct((NPAD, D_OUT), jnp.float32),
    )(agg2, agg2, g2, deg2d, b2row)


def kernel(x, edge_index, W1, b1, W2, b2):
    x_pad = jnp.zeros((NPAD, D_IN), jnp.float32).at[:N].set(x)
    pad = jnp.full((2, EPAD - E), N, jnp.int32)
    ei = jnp.concatenate([edge_index, pad], axis=1)
    dst_r = ei[0].reshape(NSUB, NCH1, CK)
    esd = jnp.stack(
        [ei[1].reshape(TOTCH, CK), ei[0].reshape(TOTCH, CK)], axis=1)

    deg = _deg_kernel(dst_r)
    deg2d = deg[:, None]

    g1 = _mm1(x_pad, W1, deg2d)
    agg1 = _agg_l1(g1, esd)
    g2 = _layer2(agg1, g1, deg2d, b1.reshape(1, D_H), W2)
    agg2 = _agg_l2(g2, esd)
    out = _final(agg2, g2, deg2d, b2.reshape(1, D_OUT))
    return out[:N]
